# Initial kernel scaffold; baseline (speedup 1.0000x reference)
import jax
import jax.numpy as jnp
from jax.experimental import pallas as pl


def kernel(vertex_attr, edge_index, edge_attr, g, batch, W1, b1, W2, b2, W3, b3):
    raise NotImplementedError("write your pallas kernel here")


# speedup vs baseline: 2.2128x; 2.2128x over previous
import functools

import jax
import jax.numpy as jnp
from jax import lax
from jax.experimental import pallas as pl
from jax.experimental.pallas import tpu as pltpu
from jax.experimental.pallas import tpu_sc as plsc

_N_NODES = 100000
_N_PAD = 100352
_N_EDGES = 6400000

_NC = 2
_NS = 16
_L = 16

_EDGES_PER_TILE = _N_EDGES // _NS
_CHUNK = 4000
_N_CHUNKS = _EDGES_PER_TILE // _CHUNK

_GDN = lax.GatherDimensionNumbers(
    offset_dims=(), collapsed_slice_dims=(0,), start_index_map=(0,))


def _vgather(v, idx):
  return lax.gather(v, idx[:, None], _GDN, (1,),
                    mode=lax.GatherScatterMode.PROMISE_IN_BOUNDS)


def _sc_scatter_body(dst_hbm, val_hbm, out_hbm, acc, idxb, valb):
  c = lax.axis_index("c")
  s = lax.axis_index("s")

  zero16 = jnp.zeros((_L,), jnp.float32)
  def _zero(j, carry):
    acc[pl.ds(j * _L, _L)] = zero16
    return carry
  lax.fori_loop(0, _N_PAD // _L, _zero, 0)

  lanes = jnp.arange(_L, dtype=jnp.int32)
  gidx = (2 * lanes + c) & (_L - 1)
  low_half = lanes < 8

  base_edge = s * _EDGES_PER_TILE

  def _chunk(k, carry):
    eb = base_edge + k * _CHUNK
    pltpu.sync_copy(dst_hbm.at[pl.ds(eb, _CHUNK)], idxb)
    pltpu.sync_copy(val_hbm.at[pl.ds(2 * eb, 2 * _CHUNK)], valb)

    def _inner(i, icarry):
      d = idxb[pl.ds(i * _L, _L)]
      v0 = valb[pl.ds(i * 2 * _L, _L)]
      v1 = valb[pl.ds(i * 2 * _L + _L, _L)]
      col = jnp.where(low_half, _vgather(v0, gidx), _vgather(v1, gidx))
      plsc.addupdate_scatter(acc, [d], col)
      return icarry

    lax.fori_loop(0, _CHUNK // _L, _inner, 0)
    return carry

  lax.fori_loop(0, _N_CHUNKS, _chunk, 0)

  pltpu.sync_copy(acc, out_hbm.at[c * _NS + s])


_sc_scatter = functools.partial(
    pl.kernel,
    out_type=jax.ShapeDtypeStruct((_NC * _NS, _N_PAD), jnp.float32),
    mesh=plsc.VectorSubcoreMesh(core_axis_name="c", subcore_axis_name="s"),
    compiler_params=pltpu.CompilerParams(needs_layout_passes=False),
    scratch_types=[
        pltpu.VMEM((_N_PAD,), jnp.float32),
        pltpu.VMEM((_CHUNK,), jnp.int32),
        pltpu.VMEM((2 * _CHUNK,), jnp.float32),
    ],
)(_sc_scatter_body)


_ROWS = 12544


def _mlp_body(vref, pref, w1v_ref, w1a_ref, b1_ref, w2_ref, b2_ref,
              w3_ref, b3_ref, oref):
  v = vref[...]
  p = pref[...]
  h = jnp.dot(v, w1v_ref[...], preferred_element_type=jnp.float32)
  h = h + lax.dot_general(p, w1a_ref[...], (((0,), (0,)), ((), ())),
                          preferred_element_type=jnp.float32)
  h = jnp.maximum(h + b1_ref[...], 0.0)
  h = jnp.maximum(
      jnp.dot(h, w2_ref[...], preferred_element_type=jnp.float32)
      + b2_ref[...], 0.0)
  oref[...] = (jnp.dot(h, w3_ref[...], preferred_element_type=jnp.float32)
               + b3_ref[...])


def _mlp(vertex_attr, partials, w1v, w1a, b1, w2t, b2, w3t, b3):
  grid = _N_PAD // _ROWS
  return pl.pallas_call(
      _mlp_body,
      out_shape=jax.ShapeDtypeStruct((_N_PAD, 1), jnp.float32),
      grid=(grid,),
      in_specs=[
          pl.BlockSpec((_ROWS, 3), lambda i: (i, 0)),
          pl.BlockSpec((_NC * _NS, _ROWS), lambda i: (0, i)),
          pl.BlockSpec((3, 50), lambda i: (0, 0)),
          pl.BlockSpec((_NC * _NS, 50), lambda i: (0, 0)),
          pl.BlockSpec((1, 50), lambda i: (0, 0)),
          pl.BlockSpec((50, 20), lambda i: (0, 0)),
          pl.BlockSpec((1, 20), lambda i: (0, 0)),
          pl.BlockSpec((20, 1), lambda i: (0, 0)),
          pl.BlockSpec((1, 1), lambda i: (0, 0)),
      ],
      out_specs=pl.BlockSpec((_ROWS, 1), lambda i: (i, 0)),
  )(vertex_attr, partials, w1v, w1a, b1, w2t, b2, w3t, b3)


def kernel(vertex_attr, edge_index, edge_attr, g, batch, W1, b1, W2, b2,
           W3, b3):
  del g, batch
  dst = edge_index[1]
  vals = edge_attr.reshape(-1)

  partials = _sc_scatter(dst, vals)

  w1t = W1.T
  w1v = w1t[:3]
  w1a = jnp.concatenate([
      jnp.broadcast_to(w1t[3:4], (_NS, 50)),
      jnp.broadcast_to(w1t[4:5], (_NS, 50)),
  ], axis=0)

  vpad = jnp.pad(vertex_attr, ((0, _N_PAD - _N_NODES), (0, 0)))
  out = _mlp(vpad, partials, w1v, w1a, b1.reshape(1, 50),
             W2.T, b2.reshape(1, 20), W3.T, b3.reshape(1, 1))
  return out[:_N_NODES]


# speedup vs baseline: 38.6981x; 17.4879x over previous
import functools

import jax
import jax.numpy as jnp
from jax import lax
from jax.experimental import pallas as pl
from jax.experimental.pallas import tpu as pltpu
from jax.experimental.pallas import tpu_sc as plsc

_N_NODES = 100000
_N_PAD = 100352
_N_EDGES = 6400000

_NC = 2
_NS = 16
_L = 16

_N_ROWS = _N_EDGES // 64
_CHUNK_ROWS = 40
_CHUNK_BLOCKS = _CHUNK_ROWS // 2
_N_CHUNKS = _N_ROWS // _CHUNK_ROWS
_BASE_CHUNKS = _N_CHUNKS // _NS
_EXTRA = _N_CHUNKS - _BASE_CHUNKS * _NS


def _sc_scatter_body(idx_hbm, val_hbm, out_hbm, acc, idxb, valb):
  c = lax.axis_index("c")
  s = lax.axis_index("s")

  zero16 = jnp.zeros((_L,), jnp.float32)
  def _zero(j, carry):
    acc[pl.ds(j * _L, _L)] = zero16
    return carry
  lax.fori_loop(0, _N_PAD // _L, _zero, 0)

  n_chunks = _BASE_CHUNKS + jnp.where(s < _EXTRA, 1, 0)

  def _chunk(k, carry):
    row0 = (s + k * _NS) * _CHUNK_ROWS
    pltpu.sync_copy(idx_hbm.at[pl.ds(row0, _CHUNK_ROWS)], idxb)
    pltpu.sync_copy(val_hbm.at[pl.ds(row0, _CHUNK_ROWS)], valb)

    def _blk(j, jcarry):
      for t in range(128 // _L):
        d = idxb[2 * j + 1, pl.ds(t * _L, _L)]
        v = valb[2 * j + c, pl.ds(t * _L, _L)]
        plsc.addupdate_scatter(acc, [d], v)
      return jcarry

    lax.fori_loop(0, _CHUNK_BLOCKS, _blk, 0)
    return carry

  lax.fori_loop(0, n_chunks, _chunk, 0)

  pltpu.sync_copy(acc, out_hbm.at[c * _NS + s])


_sc_scatter = functools.partial(
    pl.kernel,
    out_type=jax.ShapeDtypeStruct((_NC * _NS, _N_PAD), jnp.float32),
    mesh=plsc.VectorSubcoreMesh(core_axis_name="c", subcore_axis_name="s"),
    compiler_params=pltpu.CompilerParams(needs_layout_passes=False),
    scratch_types=[
        pltpu.VMEM((_N_PAD,), jnp.float32),
        pltpu.VMEM((_CHUNK_ROWS, 128), jnp.int32),
        pltpu.VMEM((_CHUNK_ROWS, 128), jnp.float32),
    ],
)(_sc_scatter_body)


_ROWS = 12544


def _mlp_body(vt_ref, p_ref, w1v_ref, w1a_ref, b1_ref, w2_ref, b2_ref,
              w3_ref, b3_ref, oref):
  vt = vt_ref[...]
  p = p_ref[...]
  h = lax.dot_general(vt, w1v_ref[...], (((0,), (0,)), ((), ())),
                      preferred_element_type=jnp.float32)
  h = h + lax.dot_general(p, w1a_ref[...], (((0,), (0,)), ((), ())),
                          preferred_element_type=jnp.float32)
  h = jnp.maximum(h + b1_ref[...], 0.0)
  h = jnp.maximum(
      jnp.dot(h, w2_ref[...], preferred_element_type=jnp.float32)
      + b2_ref[...], 0.0)
  o = lax.dot_general(w3_ref[...], h, (((1,), (1,)), ((), ())),
                      preferred_element_type=jnp.float32)
  oref[...] = o + b3_ref[...]


def _mlp(vt, partials, w1v, w1a, b1, w2t, b2, w3r, b3):
  grid = _N_PAD // _ROWS
  return pl.pallas_call(
      _mlp_body,
      out_shape=jax.ShapeDtypeStruct((1, _N_NODES), jnp.float32),
      grid=(grid,),
      in_specs=[
          pl.BlockSpec((3, _ROWS), lambda i: (0, i)),
          pl.BlockSpec((_NC * _NS, _ROWS), lambda i: (0, i)),
          pl.BlockSpec((3, 50), lambda i: (0, 0)),
          pl.BlockSpec((_NC * _NS, 50), lambda i: (0, 0)),
          pl.BlockSpec((1, 50), lambda i: (0, 0)),
          pl.BlockSpec((50, 20), lambda i: (0, 0)),
          pl.BlockSpec((1, 20), lambda i: (0, 0)),
          pl.BlockSpec((1, 20), lambda i: (0, 0)),
          pl.BlockSpec((1, 1), lambda i: (0, 0)),
      ],
      out_specs=pl.BlockSpec((1, _ROWS), lambda i: (0, i)),
  )(vt, partials, w1v, w1a, b1, w2t, b2, w3r, b3)


def kernel(vertex_attr, edge_index, edge_attr, g, batch, W1, b1, W2, b2,
           W3, b3):
  del g, batch
  nb = _N_EDGES // 128
  idxv = edge_index.reshape(2, nb, 128).transpose(1, 0, 2)
  idxv = idxv.reshape(_N_ROWS, 128)
  vals = edge_attr.reshape(nb, 128, 2).transpose(0, 2, 1)
  vals = vals.reshape(_N_ROWS, 128)

  partials = _sc_scatter(idxv, vals)

  w1t = W1.T
  w1v = w1t[:3]
  w1a = jnp.concatenate([
      jnp.broadcast_to(w1t[3:4], (_NS, 50)),
      jnp.broadcast_to(w1t[4:5], (_NS, 50)),
  ], axis=0)

  out = _mlp(vertex_attr.T, partials, w1v, w1a, b1.reshape(1, 50),
             W2.T, b2.reshape(1, 20), W3.reshape(1, 20),
             b3.reshape(1, 1))
  return out.reshape(_N_NODES, 1)


# speedup vs baseline: 98.7145x; 2.5509x over previous
import functools

import jax
import jax.numpy as jnp
from jax import lax
from jax.experimental import pallas as pl
from jax.experimental.pallas import tpu as pltpu
from jax.experimental.pallas import tpu_sc as plsc

_N_NODES = 100000
_N_PAD = 100352
_N_EDGES = 6400000

_NC = 2
_NS = 16
_L = 16

_N_ROWS = _N_EDGES // 64
_CHUNK_ROWS = 40
_CHUNK_BLOCKS = _CHUNK_ROWS // 2
_N_CHUNKS = _N_ROWS // _CHUNK_ROWS
_BASE_CHUNKS = _N_CHUNKS // _NS
_EXTRA = _N_CHUNKS - _BASE_CHUNKS * _NS


_PAIRS = _BASE_CHUNKS // 2


def _sc_scatter_body(idx_hbm, val_hbm, out_hbm, acc, idx0, val0, idx1, val1,
                     sem0, sem1):
  c = lax.axis_index("c")
  s = lax.axis_index("s")

  zero16 = jnp.zeros((_L,), jnp.float32)
  @plsc.parallel_loop(0, _N_PAD // _L, unroll=4)
  def _zero(j):
    acc[pl.ds(j * _L, _L)] = zero16

  def _issue(k, ib, vb, sem):
    row0 = (s + k * _NS) * _CHUNK_ROWS
    pltpu.async_copy(idx_hbm.at[pl.ds(row0, _CHUNK_ROWS)], ib, sem)
    pltpu.async_copy(val_hbm.at[pl.ds(row0, _CHUNK_ROWS)], vb, sem)

  def _wait(ib, vb, sem):
    pltpu.make_async_copy(idx_hbm.at[pl.ds(0, _CHUNK_ROWS)], ib, sem).wait()
    pltpu.make_async_copy(val_hbm.at[pl.ds(0, _CHUNK_ROWS)], vb, sem).wait()

  def _compute(ib, vb):
    @plsc.parallel_loop(0, _CHUNK_BLOCKS)
    def _blk(j):
      for t in range(128 // _L):
        d = ib[2 * j + 1, pl.ds(t * _L, _L)]
        v = vb[2 * j + c, pl.ds(t * _L, _L)]
        plsc.addupdate_scatter(acc, [d], v)

  _issue(0, idx0, val0, sem0)

  def _pair(k, carry):
    _issue(2 * k + 1, idx1, val1, sem1)
    _wait(idx0, val0, sem0)
    _compute(idx0, val0)

    @pl.when(k < _PAIRS - 1)
    def _():
      _issue(2 * k + 2, idx0, val0, sem0)

    _wait(idx1, val1, sem1)
    _compute(idx1, val1)
    return carry

  lax.fori_loop(0, _PAIRS, _pair, 0)

  @pl.when(s < _EXTRA)
  def _():
    _issue(_BASE_CHUNKS, idx0, val0, sem0)
    _wait(idx0, val0, sem0)
    _compute(idx0, val0)

  pltpu.sync_copy(acc, out_hbm.at[c * _NS + s])


_sc_scatter = functools.partial(
    pl.kernel,
    out_type=jax.ShapeDtypeStruct((_NC * _NS, _N_PAD), jnp.float32),
    mesh=plsc.VectorSubcoreMesh(core_axis_name="c", subcore_axis_name="s"),
    compiler_params=pltpu.CompilerParams(needs_layout_passes=False),
    scratch_types=[
        pltpu.VMEM((_N_PAD,), jnp.float32),
        pltpu.VMEM((_CHUNK_ROWS, 128), jnp.int32),
        pltpu.VMEM((_CHUNK_ROWS, 128), jnp.float32),
        pltpu.VMEM((_CHUNK_ROWS, 128), jnp.int32),
        pltpu.VMEM((_CHUNK_ROWS, 128), jnp.float32),
        pltpu.SemaphoreType.DMA,
        pltpu.SemaphoreType.DMA,
    ],
)(_sc_scatter_body)


_ROWS = 12544


def _mlp_body(vt_ref, p_ref, w1v_ref, w1a_ref, b1_ref, w2_ref, b2_ref,
              w3_ref, b3_ref, oref):
  vt = vt_ref[...]
  p = p_ref[...]
  h = lax.dot_general(vt, w1v_ref[...], (((0,), (0,)), ((), ())),
                      preferred_element_type=jnp.float32)
  h = h + lax.dot_general(p, w1a_ref[...], (((0,), (0,)), ((), ())),
                          preferred_element_type=jnp.float32)
  h = jnp.maximum(h + b1_ref[...], 0.0)
  h = jnp.maximum(
      jnp.dot(h, w2_ref[...], preferred_element_type=jnp.float32)
      + b2_ref[...], 0.0)
  o = lax.dot_general(w3_ref[...], h, (((1,), (1,)), ((), ())),
                      preferred_element_type=jnp.float32)
  oref[...] = o + b3_ref[...]


def _mlp(vt, partials, w1v, w1a, b1, w2t, b2, w3r, b3):
  grid = _N_PAD // _ROWS
  return pl.pallas_call(
      _mlp_body,
      out_shape=jax.ShapeDtypeStruct((1, _N_NODES), jnp.float32),
      grid=(grid,),
      in_specs=[
          pl.BlockSpec((3, _ROWS), lambda i: (0, i)),
          pl.BlockSpec((_NC * _NS, _ROWS), lambda i: (0, i)),
          pl.BlockSpec((3, 50), lambda i: (0, 0)),
          pl.BlockSpec((_NC * _NS, 50), lambda i: (0, 0)),
          pl.BlockSpec((1, 50), lambda i: (0, 0)),
          pl.BlockSpec((50, 20), lambda i: (0, 0)),
          pl.BlockSpec((1, 20), lambda i: (0, 0)),
          pl.BlockSpec((1, 20), lambda i: (0, 0)),
          pl.BlockSpec((1, 1), lambda i: (0, 0)),
      ],
      out_specs=pl.BlockSpec((1, _ROWS), lambda i: (0, i)),
  )(vt, partials, w1v, w1a, b1, w2t, b2, w3r, b3)


def kernel(vertex_attr, edge_index, edge_attr, g, batch, W1, b1, W2, b2,
           W3, b3):
  del g, batch
  nb = _N_EDGES // 128
  idxv = edge_index.reshape(2, nb, 128).transpose(1, 0, 2)
  idxv = idxv.reshape(_N_ROWS, 128)
  vals = edge_attr.reshape(nb, 128, 2).transpose(0, 2, 1)
  vals = vals.reshape(_N_ROWS, 128)

  partials = _sc_scatter(idxv, vals)

  w1t = W1.T
  w1v = w1t[:3]
  w1a = jnp.concatenate([
      jnp.broadcast_to(w1t[3:4], (_NS, 50)),
      jnp.broadcast_to(w1t[4:5], (_NS, 50)),
  ], axis=0)

  out = _mlp(vertex_attr.T, partials, w1v, w1a, b1.reshape(1, 50),
             W2.T, b2.reshape(1, 20), W3.reshape(1, 20),
             b3.reshape(1, 1))
  return out.reshape(_N_NODES, 1)


# speedup vs baseline: 98.9229x; 1.0021x over previous
import functools

import jax
import jax.numpy as jnp
from jax import lax
from jax.experimental import pallas as pl
from jax.experimental.pallas import tpu as pltpu
from jax.experimental.pallas import tpu_sc as plsc

_N_NODES = 100000
_N_PAD = 100352
_N_EDGES = 6400000

_NC = 2
_NS = 16
_L = 16

_N_ROWS = _N_EDGES // 64
_CHUNK_ROWS = 40
_CHUNK_BLOCKS = _CHUNK_ROWS // 2
_N_CHUNKS = _N_ROWS // _CHUNK_ROWS
_BASE_CHUNKS = _N_CHUNKS // _NS
_EXTRA = _N_CHUNKS - _BASE_CHUNKS * _NS


_PAIRS = _BASE_CHUNKS // 2


def _sc_scatter_body(idx_hbm, val_hbm, out_hbm, acc, idx0, val0, idx1, val1,
                     sem0, sem1):
  c = lax.axis_index("c")
  s = lax.axis_index("s")

  zero16 = jnp.zeros((_L,), jnp.float32)
  @plsc.parallel_loop(0, _N_PAD // _L, unroll=4)
  def _zero(j):
    acc[pl.ds(j * _L, _L)] = zero16

  def _issue(k, ib, vb, sem):
    row0 = (s + k * _NS) * _CHUNK_ROWS
    pltpu.async_copy(idx_hbm.at[pl.ds(row0, _CHUNK_ROWS)], ib, sem)
    pltpu.async_copy(val_hbm.at[pl.ds(row0, _CHUNK_ROWS)], vb, sem)

  def _wait(ib, vb, sem):
    pltpu.make_async_copy(idx_hbm.at[pl.ds(0, _CHUNK_ROWS)], ib, sem).wait()
    pltpu.make_async_copy(val_hbm.at[pl.ds(0, _CHUNK_ROWS)], vb, sem).wait()

  def _compute(ib, vb):
    @plsc.parallel_loop(0, _CHUNK_BLOCKS, unroll=2)
    def _blk(j):
      for t in range(128 // _L):
        d = ib[2 * j + 1, pl.ds(t * _L, _L)]
        v = vb[2 * j + c, pl.ds(t * _L, _L)]
        plsc.addupdate_scatter(acc, [d], v)

  _issue(0, idx0, val0, sem0)

  def _pair(k, carry):
    _issue(2 * k + 1, idx1, val1, sem1)
    _wait(idx0, val0, sem0)
    _compute(idx0, val0)

    @pl.when(k < _PAIRS - 1)
    def _():
      _issue(2 * k + 2, idx0, val0, sem0)

    _wait(idx1, val1, sem1)
    _compute(idx1, val1)
    return carry

  lax.fori_loop(0, _PAIRS, _pair, 0)

  @pl.when(s < _EXTRA)
  def _():
    _issue(_BASE_CHUNKS, idx0, val0, sem0)
    _wait(idx0, val0, sem0)
    _compute(idx0, val0)

  pltpu.sync_copy(acc, out_hbm.at[c * _NS + s])


_sc_scatter = functools.partial(
    pl.kernel,
    out_type=jax.ShapeDtypeStruct((_NC * _NS, _N_PAD), jnp.float32),
    mesh=plsc.VectorSubcoreMesh(core_axis_name="c", subcore_axis_name="s"),
    compiler_params=pltpu.CompilerParams(needs_layout_passes=False),
    scratch_types=[
        pltpu.VMEM((_N_PAD,), jnp.float32),
        pltpu.VMEM((_CHUNK_ROWS, 128), jnp.int32),
        pltpu.VMEM((_CHUNK_ROWS, 128), jnp.float32),
        pltpu.VMEM((_CHUNK_ROWS, 128), jnp.int32),
        pltpu.VMEM((_CHUNK_ROWS, 128), jnp.float32),
        pltpu.SemaphoreType.DMA,
        pltpu.SemaphoreType.DMA,
    ],
)(_sc_scatter_body)


_ROWS = 12544


def _mlp_body(vt_ref, p_ref, w1v_ref, w1a_ref, b1_ref, w2_ref, b2_ref,
              w3_ref, b3_ref, oref):
  vt = vt_ref[...]
  p = p_ref[...]
  h = lax.dot_general(vt, w1v_ref[...], (((0,), (0,)), ((), ())),
                      preferred_element_type=jnp.float32)
  h = h + lax.dot_general(p, w1a_ref[...], (((0,), (0,)), ((), ())),
                          preferred_element_type=jnp.float32)
  h = jnp.maximum(h + b1_ref[...], 0.0)
  h = jnp.maximum(
      jnp.dot(h, w2_ref[...], preferred_element_type=jnp.float32)
      + b2_ref[...], 0.0)
  o = lax.dot_general(w3_ref[...], h, (((1,), (1,)), ((), ())),
                      preferred_element_type=jnp.float32)
  oref[...] = o + b3_ref[...]


def _mlp(vt, partials, w1v, w1a, b1, w2t, b2, w3r, b3):
  grid = _N_PAD // _ROWS
  return pl.pallas_call(
      _mlp_body,
      out_shape=jax.ShapeDtypeStruct((1, _N_NODES), jnp.float32),
      grid=(grid,),
      in_specs=[
          pl.BlockSpec((3, _ROWS), lambda i: (0, i)),
          pl.BlockSpec((_NC * _NS, _ROWS), lambda i: (0, i)),
          pl.BlockSpec((3, 50), lambda i: (0, 0)),
          pl.BlockSpec((_NC * _NS, 50), lambda i: (0, 0)),
          pl.BlockSpec((1, 50), lambda i: (0, 0)),
          pl.BlockSpec((50, 20), lambda i: (0, 0)),
          pl.BlockSpec((1, 20), lambda i: (0, 0)),
          pl.BlockSpec((1, 20), lambda i: (0, 0)),
          pl.BlockSpec((1, 1), lambda i: (0, 0)),
      ],
      out_specs=pl.BlockSpec((1, _ROWS), lambda i: (0, i)),
  )(vt, partials, w1v, w1a, b1, w2t, b2, w3r, b3)


def kernel(vertex_attr, edge_index, edge_attr, g, batch, W1, b1, W2, b2,
           W3, b3):
  del g, batch
  nb = _N_EDGES // 128
  idxv = edge_index.reshape(2, nb, 128).transpose(1, 0, 2)
  idxv = idxv.reshape(_N_ROWS, 128)
  vals = edge_attr.reshape(nb, 128, 2).transpose(0, 2, 1)
  vals = vals.reshape(_N_ROWS, 128)

  partials = _sc_scatter(idxv, vals)

  w1t = W1.T
  w1v = w1t[:3]
  w1a = jnp.concatenate([
      jnp.broadcast_to(w1t[3:4], (_NS, 50)),
      jnp.broadcast_to(w1t[4:5], (_NS, 50)),
  ], axis=0)

  out = _mlp(vertex_attr.T, partials, w1v, w1a, b1.reshape(1, 50),
             W2.T, b2.reshape(1, 20), W3.reshape(1, 20),
             b3.reshape(1, 1))
  return out.reshape(_N_NODES, 1)


# speedup vs baseline: 103.5461x; 1.0467x over previous
import functools

import jax
import jax.numpy as jnp
from jax import lax
from jax.experimental import pallas as pl
from jax.experimental.pallas import tpu as pltpu
from jax.experimental.pallas import tpu_sc as plsc

_N_NODES = 100000
_N_PAD = 100352
_N_EDGES = 6400000

_NC = 2
_NS = 16
_L = 16

_N_ROWS = _N_EDGES // 64
_CB = 16
_N_CHUNKS = (_N_EDGES // 128) // _CB
_BASE_CHUNKS = _N_CHUNKS // _NS
_EXTRA = _N_CHUNKS - _BASE_CHUNKS * _NS
_PAIRS = _BASE_CHUNKS // 2


def _sc_scatter_body(idx_hbm, val_hbm, out_hbm, acc, idx0, val0, idx1, val1,
                     sem0, sem1):
  c = lax.axis_index("c")
  s = lax.axis_index("s")

  zero16 = jnp.zeros((_L,), jnp.float32)
  @plsc.parallel_loop(0, _N_PAD // _L, unroll=4)
  def _zero(j):
    acc[pl.ds(j * _L, _L)] = zero16

  iota2 = 2 * jnp.arange(_L, dtype=jnp.int32)

  def _issue(k, ib, vb, sem):
    rows = 2 * ((s + k * _NS) * _CB) + iota2
    pltpu.async_copy(idx_hbm.at[rows + 1], ib, sem)
    pltpu.async_copy(val_hbm.at[rows + c], vb, sem)

  def _wait(ib, vb, sem):
    pltpu.make_async_copy(idx_hbm.at[iota2], ib, sem).wait()
    pltpu.make_async_copy(val_hbm.at[iota2], vb, sem).wait()

  def _compute(ib, vb):
    @plsc.parallel_loop(0, _CB, unroll=2)
    def _blk(j):
      for t in range(128 // _L):
        d = ib[j, pl.ds(t * _L, _L)]
        v = vb[j, pl.ds(t * _L, _L)]
        plsc.addupdate_scatter(acc, [d], v)

  _issue(0, idx0, val0, sem0)

  def _pair(k, carry):
    _issue(2 * k + 1, idx1, val1, sem1)
    _wait(idx0, val0, sem0)
    _compute(idx0, val0)
    _issue(2 * k + 2, idx0, val0, sem0)
    _wait(idx1, val1, sem1)
    _compute(idx1, val1)
    return carry

  lax.fori_loop(0, _PAIRS, _pair, 0)

  @pl.when(s < _EXTRA)
  def _():
    _issue(_BASE_CHUNKS, idx1, val1, sem1)

  _wait(idx0, val0, sem0)
  _compute(idx0, val0)

  @pl.when(s < _EXTRA)
  def _():
    _wait(idx1, val1, sem1)
    _compute(idx1, val1)

  pltpu.sync_copy(acc, out_hbm.at[c * _NS + s])


_sc_scatter = functools.partial(
    pl.kernel,
    out_type=jax.ShapeDtypeStruct((_NC * _NS, _N_PAD), jnp.float32),
    mesh=plsc.VectorSubcoreMesh(core_axis_name="c", subcore_axis_name="s"),
    compiler_params=pltpu.CompilerParams(needs_layout_passes=False),
    scratch_types=[
        pltpu.VMEM((_N_PAD,), jnp.float32),
        pltpu.VMEM((_CB, 128), jnp.int32),
        pltpu.VMEM((_CB, 128), jnp.float32),
        pltpu.VMEM((_CB, 128), jnp.int32),
        pltpu.VMEM((_CB, 128), jnp.float32),
        pltpu.SemaphoreType.DMA,
        pltpu.SemaphoreType.DMA,
    ],
)(_sc_scatter_body)


_ROWS = 12544


def _mlp_body(vt_ref, p_ref, w1v_ref, w1a_ref, b1_ref, w2_ref, b2_ref,
              w3_ref, b3_ref, oref):
  vt = vt_ref[...]
  p = p_ref[...]
  h = lax.dot_general(vt, w1v_ref[...], (((0,), (0,)), ((), ())),
                      preferred_element_type=jnp.float32)
  h = h + lax.dot_general(p, w1a_ref[...], (((0,), (0,)), ((), ())),
                          preferred_element_type=jnp.float32)
  h = jnp.maximum(h + b1_ref[...], 0.0)
  h = jnp.maximum(
      jnp.dot(h, w2_ref[...], preferred_element_type=jnp.float32)
      + b2_ref[...], 0.0)
  o = lax.dot_general(w3_ref[...], h, (((1,), (1,)), ((), ())),
                      preferred_element_type=jnp.float32)
  oref[...] = o + b3_ref[...]


def _mlp(vt, partials, w1v, w1a, b1, w2t, b2, w3r, b3):
  grid = _N_PAD // _ROWS
  return pl.pallas_call(
      _mlp_body,
      out_shape=jax.ShapeDtypeStruct((1, _N_NODES), jnp.float32),
      grid=(grid,),
      in_specs=[
          pl.BlockSpec((3, _ROWS), lambda i: (0, i)),
          pl.BlockSpec((_NC * _NS, _ROWS), lambda i: (0, i)),
          pl.BlockSpec((3, 50), lambda i: (0, 0)),
          pl.BlockSpec((_NC * _NS, 50), lambda i: (0, 0)),
          pl.BlockSpec((1, 50), lambda i: (0, 0)),
          pl.BlockSpec((50, 20), lambda i: (0, 0)),
          pl.BlockSpec((1, 20), lambda i: (0, 0)),
          pl.BlockSpec((1, 20), lambda i: (0, 0)),
          pl.BlockSpec((1, 1), lambda i: (0, 0)),
      ],
      out_specs=pl.BlockSpec((1, _ROWS), lambda i: (0, i)),
  )(vt, partials, w1v, w1a, b1, w2t, b2, w3r, b3)


def kernel(vertex_attr, edge_index, edge_attr, g, batch, W1, b1, W2, b2,
           W3, b3):
  del g, batch
  nb = _N_EDGES // 128
  idxv = edge_index.reshape(2, nb, 128).transpose(1, 0, 2)
  idxv = idxv.reshape(_N_ROWS, 128)
  vals = edge_attr.reshape(nb, 128, 2).transpose(0, 2, 1)
  vals = vals.reshape(_N_ROWS, 128)

  partials = _sc_scatter(idxv, vals)

  w1t = W1.T
  w1v = w1t[:3]
  w1a = jnp.concatenate([
      jnp.broadcast_to(w1t[3:4], (_NS, 50)),
      jnp.broadcast_to(w1t[4:5], (_NS, 50)),
  ], axis=0)

  out = _mlp(vertex_attr.T, partials, w1v, w1a, b1.reshape(1, 50),
             W2.T, b2.reshape(1, 20), W3.reshape(1, 20),
             b3.reshape(1, 1))
  return out.reshape(_N_NODES, 1)
